# Initial kernel scaffold; baseline (speedup 1.0000x reference)
import jax
import jax.numpy as jnp
from jax.experimental import pallas as pl


def kernel(x, edge_index, edge_attr, batch, W0, b0, W1, b1, W2, b2, g0, be0, g1, be1, g2, be2, Wl, bl):
    raise NotImplementedError("write your pallas kernel here")


# speedup vs baseline: 3.5020x; 3.5020x over previous
import functools

import jax
import jax.numpy as jnp
from jax import lax
from jax.experimental import pallas as pl
from jax.experimental.pallas import tpu as pltpu
from jax.experimental.pallas import tpu_sc as plsc

N = 10000
E = 320000
H = 128
G = 64
NPAD = 10240
NW = 32
CH = 128
NCHUNK = -(-E // (NW * CH))
EPAD = NW * CH * NCHUNK
RPT = NPAD // NW
RPS = NPAD // 16

_MESH = plsc.VectorSubcoreMesh(core_axis_name="c", subcore_axis_name="s")


def _sc_deg_body(dstP, degP, dst_v, ones_v, zbuf, acc):
    c = lax.axis_index("c")
    s = lax.axis_index("s")
    w = c * 16 + s
    zeros16 = jnp.zeros((16,), jnp.float32)
    ones16 = jnp.ones((16,), jnp.float32)

    def init_row(i, _):
        zbuf[i, pl.ds(0, 16)] = zeros16
        ones_v[i, pl.ds(0, 16)] = ones16
        return 0

    lax.fori_loop(0, CH, init_row, 0)
    for t in range(RPS // CH):
        pltpu.sync_copy(zbuf, acc.at[pl.ds(s * RPS + t * CH, CH)])
    plsc.subcore_barrier()
    pltpu.sync_copy(dstP.at[w], dst_v)

    def body(j, _):
        pltpu.sync_copy(ones_v, acc.at[dst_v.at[j]], add=True)
        return 0

    lax.fori_loop(0, NCHUNK, body, 0)
    plsc.subcore_barrier()
    for t in range(RPS // CH):
        pltpu.sync_copy(acc.at[pl.ds(s * RPS + t * CH, CH)],
                        degP.at[c].at[pl.ds(s * RPS + t * CH, CH)])


def _sc_deg(dstP):
    return pl.kernel(
        _sc_deg_body,
        out_type=jax.ShapeDtypeStruct((2, NPAD, 16), jnp.float32),
        mesh=_MESH,
        scratch_types=[
            pltpu.VMEM((NCHUNK, CH), jnp.int32),
            pltpu.VMEM((CH, 16), jnp.float32),
            pltpu.VMEM((CH, 16), jnp.float32),
            pltpu.VMEM_SHARED((NPAD, 16), jnp.float32),
        ],
    )(dstP)


def _sc_msg_body(yP, srcP, dstP, SP, src_v, dst_v, gbuf, acc, sem):
    c = lax.axis_index("c")
    s = lax.axis_index("s")
    w = c * 16 + s
    zeros16 = jnp.zeros((16,), jnp.float32)

    def zrow(i, _):
        gbuf[i // 8, pl.ds((i % 8) * 16, 16)] = zeros16
        return 0

    lax.fori_loop(0, CH * H // 16, zrow, 0)
    for t in range(RPS // CH):
        pltpu.sync_copy(gbuf, acc.at[pl.ds(s * RPS + t * CH, CH)])
    plsc.subcore_barrier()
    pltpu.sync_copy(srcP.at[w], src_v)
    pltpu.sync_copy(dstP.at[w], dst_v)

    def body(j, _):
        pltpu.async_copy(yP.at[src_v.at[j]], gbuf, sem).wait()
        pltpu.sync_copy(gbuf, acc.at[dst_v.at[j]], add=True)
        return 0

    lax.fori_loop(0, NCHUNK, body, 0)
    plsc.subcore_barrier()
    for t in range(RPS // CH):
        pltpu.sync_copy(acc.at[pl.ds(s * RPS + t * CH, CH)],
                        SP.at[c].at[pl.ds(s * RPS + t * CH, CH)])


def _sc_msg(yP, srcP, dstP):
    return pl.kernel(
        _sc_msg_body,
        out_type=jax.ShapeDtypeStruct((2, NPAD, H), jnp.float32),
        mesh=_MESH,
        scratch_types=[
            pltpu.VMEM((NCHUNK, CH), jnp.int32),
            pltpu.VMEM((NCHUNK, CH), jnp.int32),
            pltpu.VMEM((CH, H), jnp.float32),
            pltpu.VMEM_SHARED((NPAD, H), jnp.float32),
            pltpu.SemaphoreType.DMA,
        ],
    )(yP, srcP, dstP)


def _sc_pool_body(hflat, batchP, mxP, smP, cntP, hbuf, bbuf, mxl, sml, cnl):
    c = lax.axis_index("c")
    s = lax.axis_index("s")
    w = c * 16 + s
    zeros16 = jnp.zeros((16,), jnp.float32)
    ninf16 = jnp.full((16,), -jnp.inf, jnp.float32)
    ones16 = jnp.ones((16,), jnp.float32)

    def init_mx(i, _):
        mxl[pl.ds(i * 16, 16)] = ninf16
        sml[pl.ds(i * 16, 16)] = zeros16
        return 0

    lax.fori_loop(0, (G + 1) * H // 16, init_mx, 0)

    def init_cn(i, _):
        cnl[pl.ds(i * 16, 16)] = zeros16
        return 0

    lax.fori_loop(0, G + 1, init_cn, 0)

    pltpu.sync_copy(hflat.at[pl.ds(w * RPT * H, RPT * H)], hbuf)
    pltpu.sync_copy(batchP.at[pl.ds(w * RPT, RPT)], bbuf.at[pl.ds(0, RPT)])

    def row(r, _):
        b = bbuf[pl.ds(r, 16)][0]
        hb = r * H
        mb = b * H
        for k in range(H // 16):
            hv = hbuf[pl.ds(hb + k * 16, 16)]
            mv = mxl[pl.ds(mb + k * 16, 16)]
            mxl[pl.ds(mb + k * 16, 16)] = jnp.maximum(mv, hv)
            sv = sml[pl.ds(mb + k * 16, 16)]
            sml[pl.ds(mb + k * 16, 16)] = sv + hv
        cb = b * 16
        cnl[pl.ds(cb, 16)] = cnl[pl.ds(cb, 16)] + ones16
        return 0

    lax.fori_loop(0, RPT, row, 0)
    pltpu.sync_copy(mxl.at[pl.ds(0, G * H)], mxP.at[w])
    pltpu.sync_copy(sml.at[pl.ds(0, G * H)], smP.at[w])
    pltpu.sync_copy(cnl.at[pl.ds(0, G * 16)], cntP.at[w])


def _sc_pool(hflat, batchP):
    return pl.kernel(
        _sc_pool_body,
        out_type=(
            jax.ShapeDtypeStruct((NW, G * H), jnp.float32),
            jax.ShapeDtypeStruct((NW, G * H), jnp.float32),
            jax.ShapeDtypeStruct((NW, G * 16), jnp.float32),
        ),
        mesh=_MESH,
        scratch_types=[
            pltpu.VMEM((RPT * H,), jnp.float32),
            pltpu.VMEM((RPT + 16,), jnp.int32),
            pltpu.VMEM(((G + 1) * H,), jnp.float32),
            pltpu.VMEM(((G + 1) * H,), jnp.float32),
            pltpu.VMEM(((G + 1) * 16,), jnp.float32),
        ],
    )(hflat, batchP)


_NBLK = NPAD // 8
_LAST = N // 8 - 1


def _dinv_blk(deg_ref):
    deg = deg_ref[0] + deg_ref[1]
    return lax.rsqrt(deg[:, 0:1] + 1.0)


def _tc_y0_body(x_ref, w_ref, deg_ref, y_ref):
    i = pl.program_id(0)
    dinv = _dinv_blk(deg_ref)
    xw = jnp.dot(x_ref[...], w_ref[...], preferred_element_type=jnp.float32, precision=lax.Precision.HIGHEST)
    y_ref[...] = jnp.where(i <= _LAST, xw * dinv, 0.0)


def _tc_y0(x, W0, degP):
    return pl.pallas_call(
        _tc_y0_body,
        grid=(_NBLK,),
        in_specs=[
            pl.BlockSpec((8, H), lambda i: (jnp.minimum(i, _LAST), 0)),
            pl.BlockSpec((H, H), lambda i: (0, 0)),
            pl.BlockSpec((2, 8, 16), lambda i: (0, i, 0)),
        ],
        out_specs=pl.BlockSpec((8, H), lambda i: (i, 0)),
        out_shape=jax.ShapeDtypeStruct((NPAD, H), jnp.float32),
    )(x, W0, degP)


def _ln_leaky(S_ref, y_ref, deg_ref, b_ref, g_ref, be_ref):
    dinv = _dinv_blk(deg_ref)
    S = S_ref[0] + S_ref[1]
    conv = dinv * (S + y_ref[...]) + b_ref[...]
    mu = jnp.mean(conv, axis=-1, keepdims=True)
    var = jnp.mean((conv - mu) ** 2, axis=-1, keepdims=True)
    a = (conv - mu) * lax.rsqrt(var + 1e-5) * g_ref[...] + be_ref[...]
    return jnp.where(a > 0, a, 0.1 * a), dinv


def _tc_mid_body(S_ref, y_ref, deg_ref, b_ref, g_ref, be_ref, wn_ref, yo_ref):
    i = pl.program_id(0)
    h, dinv = _ln_leaky(S_ref, y_ref, deg_ref, b_ref, g_ref, be_ref)
    yn = jnp.dot(h, wn_ref[...], preferred_element_type=jnp.float32, precision=lax.Precision.HIGHEST) * dinv
    yo_ref[...] = jnp.where(i <= _LAST, yn, 0.0)


def _tc_mid(SP, y, degP, b, g, be, Wn):
    return pl.pallas_call(
        _tc_mid_body,
        grid=(_NBLK,),
        in_specs=[
            pl.BlockSpec((2, 8, H), lambda i: (0, i, 0)),
            pl.BlockSpec((8, H), lambda i: (i, 0)),
            pl.BlockSpec((2, 8, 16), lambda i: (0, i, 0)),
            pl.BlockSpec((1, H), lambda i: (0, 0)),
            pl.BlockSpec((1, H), lambda i: (0, 0)),
            pl.BlockSpec((1, H), lambda i: (0, 0)),
            pl.BlockSpec((H, H), lambda i: (0, 0)),
        ],
        out_specs=pl.BlockSpec((8, H), lambda i: (i, 0)),
        out_shape=jax.ShapeDtypeStruct((NPAD, H), jnp.float32),
    )(SP, y, degP, b, g, be, Wn)


def _tc_last_body(S_ref, y_ref, deg_ref, b_ref, g_ref, be_ref, h_ref):
    i = pl.program_id(0)
    h, _ = _ln_leaky(S_ref, y_ref, deg_ref, b_ref, g_ref, be_ref)
    h_ref[...] = jnp.where(i <= _LAST, h, 0.0)


def _tc_last(SP, y, degP, b, g, be):
    return pl.pallas_call(
        _tc_last_body,
        grid=(_NBLK,),
        in_specs=[
            pl.BlockSpec((2, 8, H), lambda i: (0, i, 0)),
            pl.BlockSpec((8, H), lambda i: (i, 0)),
            pl.BlockSpec((2, 8, 16), lambda i: (0, i, 0)),
            pl.BlockSpec((1, H), lambda i: (0, 0)),
            pl.BlockSpec((1, H), lambda i: (0, 0)),
            pl.BlockSpec((1, H), lambda i: (0, 0)),
        ],
        out_specs=pl.BlockSpec((8, H), lambda i: (i, 0)),
        out_shape=jax.ShapeDtypeStruct((NPAD, H), jnp.float32),
    )(SP, y, degP, b, g, be)


def _tc_final_body(mx_ref, sm_ref, cn_ref, wl1_ref, wl2_ref, bl_ref, o_ref):
    mx = jnp.max(mx_ref[...], axis=0)
    sm = jnp.sum(sm_ref[...], axis=0)
    cnt = jnp.sum(cn_ref[...], axis=0)
    mean = sm / jnp.maximum(cnt[:, 0:1], 1.0)
    o_ref[...] = (jnp.dot(mx, wl1_ref[...], preferred_element_type=jnp.float32, precision=lax.Precision.HIGHEST)
                  + jnp.dot(mean, wl2_ref[...], preferred_element_type=jnp.float32, precision=lax.Precision.HIGHEST)
                  + bl_ref[...])


def _tc_final(mxP, smP, cntP, Wl1, Wl2, bl):
    return pl.pallas_call(
        _tc_final_body,
        grid=(1,),
        in_specs=[
            pl.BlockSpec((NW, G, H), lambda i: (0, 0, 0)),
            pl.BlockSpec((NW, G, H), lambda i: (0, 0, 0)),
            pl.BlockSpec((NW, G, 16), lambda i: (0, 0, 0)),
            pl.BlockSpec((H, H), lambda i: (0, 0)),
            pl.BlockSpec((H, H), lambda i: (0, 0)),
            pl.BlockSpec((1, H), lambda i: (0, 0)),
        ],
        out_specs=pl.BlockSpec((G, H), lambda i: (0, 0)),
        out_shape=jax.ShapeDtypeStruct((G, H), jnp.float32),
    )(mxP, smP, cntP, Wl1, Wl2, bl)


def kernel(x, edge_index, edge_attr, batch, W0, b0, W1, b1, W2, b2,
           g0, be0, g1, be1, g2, be2, Wl, bl):
    src = edge_index[0].astype(jnp.int32)
    dst = edge_index[1].astype(jnp.int32)
    pad = N + (jnp.arange(EPAD - E, dtype=jnp.int32) % 64)
    srcP = jnp.concatenate([src, pad]).reshape(NW, NCHUNK, CH)
    dstP = jnp.concatenate([dst, pad]).reshape(NW, NCHUNK, CH)
    batchP = jnp.concatenate(
        [batch.astype(jnp.int32), jnp.full((NPAD - N,), G, jnp.int32)])

    degP = _sc_deg(dstP)
    y = _tc_y0(x, W0, degP)
    S = _sc_msg(y, srcP, dstP)
    y = _tc_mid(S, y, degP, b0.reshape(1, H), g0.reshape(1, H),
                be0.reshape(1, H), W1)
    S = _sc_msg(y, srcP, dstP)
    y = _tc_mid(S, y, degP, b1.reshape(1, H), g1.reshape(1, H),
                be1.reshape(1, H), W2)
    S = _sc_msg(y, srcP, dstP)
    h3 = _tc_last(S, y, degP, b2.reshape(1, H), g2.reshape(1, H),
                  be2.reshape(1, H))
    mxP, smP, cntP = _sc_pool(h3.reshape(-1), batchP)
    out = _tc_final(mxP.reshape(NW, G, H), smP.reshape(NW, G, H),
                    cntP.reshape(NW, G, 16), Wl[:H], Wl[H:],
                    bl.reshape(1, H))
    return out


# speedup vs baseline: 3.6529x; 1.0431x over previous
import functools

import jax
import jax.numpy as jnp
from jax import lax
from jax.experimental import pallas as pl
from jax.experimental.pallas import tpu as pltpu
from jax.experimental.pallas import tpu_sc as plsc

N = 10000
E = 320000
H = 128
G = 64
NPAD = 10240
NW = 32
CH = 128
NCHUNK = 80
EPAD = NW * CH * NCHUNK
NACC = NPAD
ZPS = NACC // 16
RPT = NPAD // NW

_MESH = plsc.VectorSubcoreMesh(core_axis_name="c", subcore_axis_name="s")


def _sc_deg_body(dstP, degP, dst_v, ones_v, zbuf, acc, semd):
    c = lax.axis_index("c")
    s = lax.axis_index("s")
    w = c * 16 + s
    zeros16 = jnp.zeros((16,), jnp.float32)
    ones16 = jnp.ones((16,), jnp.float32)

    def init_row(i, _):
        zbuf[i, pl.ds(0, 16)] = zeros16
        ones_v[i, pl.ds(0, 16)] = ones16
        return 0

    lax.fori_loop(0, CH, init_row, 0)
    for t in range(ZPS // CH):
        pltpu.sync_copy(zbuf, acc.at[pl.ds(s * ZPS + t * CH, CH)])
    plsc.subcore_barrier()
    pltpu.sync_copy(dstP.at[w], dst_v)

    def sb(b, _):
        def fire(j, _):
            pltpu.async_copy(ones_v, acc.at[dst_v.at[b * 16 + j]], semd,
                             add=True)
            return 0

        lax.fori_loop(0, 16, fire, 0)

        def drain(j, _):
            pltpu.make_async_copy(ones_v, acc.at[dst_v.at[0]], semd).wait()
            return 0

        lax.fori_loop(0, 16, drain, 0)
        return 0

    lax.fori_loop(0, NCHUNK // 16, sb, 0)
    plsc.subcore_barrier()
    for t in range(ZPS // CH):
        pltpu.sync_copy(acc.at[pl.ds(s * ZPS + t * CH, CH)],
                        degP.at[c].at[pl.ds(s * ZPS + t * CH, CH)])


def _sc_deg(dstP):
    return pl.kernel(
        _sc_deg_body,
        out_type=jax.ShapeDtypeStruct((2, NPAD, 16), jnp.float32),
        mesh=_MESH,
        scratch_types=[
            pltpu.VMEM((NCHUNK, CH), jnp.int32),
            pltpu.VMEM((CH, 16), jnp.float32),
            pltpu.VMEM((CH, 16), jnp.float32),
            pltpu.VMEM_SHARED((NACC, 16), jnp.float32),
            pltpu.SemaphoreType.DMA,
        ],
    )(dstP)


def _sc_msg_body(yP, srcP, dstP, SP, dst_v, sidx0, sidx1, gbuf0, gbuf1,
                 acc, semg0, semg1, semi0, semi1):
    c = lax.axis_index("c")
    s = lax.axis_index("s")
    w = c * 16 + s
    zeros16 = jnp.zeros((16,), jnp.float32)

    def zrow(i, _):
        gbuf0[i // 8, pl.ds((i % 8) * 16, 16)] = zeros16
        return 0

    lax.fori_loop(0, CH * H // 16, zrow, 0)
    for t in range(ZPS // CH):
        pltpu.sync_copy(gbuf0, acc.at[pl.ds(s * ZPS + t * CH, CH)])
    plsc.subcore_barrier()
    pltpu.sync_copy(dstP.at[w], dst_v)
    pltpu.sync_copy(srcP.at[w].at[0], sidx0)
    pltpu.async_copy(yP.at[sidx0], gbuf0, semg0)
    pltpu.async_copy(srcP.at[w].at[1], sidx1, semi1)

    def body(i, _):
        j0 = 2 * i
        j1 = j0 + 1
        more = i < NCHUNK // 2 - 1
        pltpu.make_async_copy(yP.at[sidx0], gbuf0, semg0).wait()
        pltpu.make_async_copy(srcP.at[w].at[0], sidx1, semi1).wait()
        pltpu.async_copy(yP.at[sidx1], gbuf1, semg1)

        @pl.when(more)
        def _():
            pltpu.async_copy(srcP.at[w].at[j0 + 2], sidx0, semi0)

        pltpu.sync_copy(gbuf0, acc.at[dst_v.at[j0]], add=True)
        pltpu.make_async_copy(yP.at[sidx1], gbuf1, semg1).wait()

        @pl.when(more)
        def _():
            pltpu.make_async_copy(srcP.at[w].at[0], sidx0, semi0).wait()
            pltpu.async_copy(yP.at[sidx0], gbuf0, semg0)
            pltpu.async_copy(srcP.at[w].at[j1 + 2], sidx1, semi1)

        pltpu.sync_copy(gbuf1, acc.at[dst_v.at[j1]], add=True)
        return 0

    lax.fori_loop(0, NCHUNK // 2, body, 0)
    plsc.subcore_barrier()
    for t in range(ZPS // CH):
        pltpu.sync_copy(acc.at[pl.ds(s * ZPS + t * CH, CH)],
                        SP.at[c].at[pl.ds(s * ZPS + t * CH, CH)])


def _sc_msg(yP, srcP, dstP):
    return pl.kernel(
        _sc_msg_body,
        out_type=jax.ShapeDtypeStruct((2, NPAD, H), jnp.float32),
        mesh=_MESH,
        scratch_types=[
            pltpu.VMEM((NCHUNK, CH), jnp.int32),
            pltpu.VMEM((CH,), jnp.int32),
            pltpu.VMEM((CH,), jnp.int32),
            pltpu.VMEM((CH, H), jnp.float32),
            pltpu.VMEM((CH, H), jnp.float32),
            pltpu.VMEM_SHARED((NACC, H), jnp.float32),
            pltpu.SemaphoreType.DMA,
            pltpu.SemaphoreType.DMA,
            pltpu.SemaphoreType.DMA,
            pltpu.SemaphoreType.DMA,
        ],
    )(yP, srcP, dstP)


def _sc_pool_body(hflat, batchP, mxP, smP, cntP, hbuf, bbuf, mxl, sml, cnl):
    c = lax.axis_index("c")
    s = lax.axis_index("s")
    w = c * 16 + s
    zeros16 = jnp.zeros((16,), jnp.float32)
    ninf16 = jnp.full((16,), -jnp.inf, jnp.float32)
    ones16 = jnp.ones((16,), jnp.float32)

    def init_mx(i, _):
        mxl[pl.ds(i * 16, 16)] = ninf16
        sml[pl.ds(i * 16, 16)] = zeros16
        return 0

    lax.fori_loop(0, (G + 1) * H // 16, init_mx, 0)

    def init_cn(i, _):
        cnl[pl.ds(i * 16, 16)] = zeros16
        return 0

    lax.fori_loop(0, G + 1, init_cn, 0)

    pltpu.sync_copy(hflat.at[pl.ds(w * RPT * H, RPT * H)], hbuf)
    pltpu.sync_copy(batchP.at[pl.ds(w * RPT, RPT)], bbuf.at[pl.ds(0, RPT)])

    def row(r, _):
        b = bbuf[pl.ds(r, 16)][0]
        hb = r * H
        mb = b * H
        for k in range(H // 16):
            hv = hbuf[pl.ds(hb + k * 16, 16)]
            mv = mxl[pl.ds(mb + k * 16, 16)]
            mxl[pl.ds(mb + k * 16, 16)] = jnp.maximum(mv, hv)
            sv = sml[pl.ds(mb + k * 16, 16)]
            sml[pl.ds(mb + k * 16, 16)] = sv + hv
        cb = b * 16
        cnl[pl.ds(cb, 16)] = cnl[pl.ds(cb, 16)] + ones16
        return 0

    lax.fori_loop(0, RPT, row, 0)
    pltpu.sync_copy(mxl.at[pl.ds(0, G * H)], mxP.at[w])
    pltpu.sync_copy(sml.at[pl.ds(0, G * H)], smP.at[w])
    pltpu.sync_copy(cnl.at[pl.ds(0, G * 16)], cntP.at[w])


def _sc_pool(hflat, batchP):
    return pl.kernel(
        _sc_pool_body,
        out_type=(
            jax.ShapeDtypeStruct((NW, G * H), jnp.float32),
            jax.ShapeDtypeStruct((NW, G * H), jnp.float32),
            jax.ShapeDtypeStruct((NW, G * 16), jnp.float32),
        ),
        mesh=_MESH,
        scratch_types=[
            pltpu.VMEM((RPT * H,), jnp.float32),
            pltpu.VMEM((RPT + 16,), jnp.int32),
            pltpu.VMEM(((G + 1) * H,), jnp.float32),
            pltpu.VMEM(((G + 1) * H,), jnp.float32),
            pltpu.VMEM(((G + 1) * 16,), jnp.float32),
        ],
    )(hflat, batchP)


_NBLK = NPAD // 8
_LAST = N // 8 - 1


def _dinv_blk(deg_ref):
    deg = deg_ref[0] + deg_ref[1]
    return lax.rsqrt(deg[:, 0:1] + 1.0)


def _tc_y0_body(x_ref, w_ref, deg_ref, y_ref):
    i = pl.program_id(0)
    dinv = _dinv_blk(deg_ref)
    xw = jnp.dot(x_ref[...], w_ref[...], preferred_element_type=jnp.float32,
                 precision=lax.Precision.HIGHEST)
    y_ref[...] = jnp.where(i <= _LAST, xw * dinv, 0.0)


def _tc_y0(x, W0, degP):
    return pl.pallas_call(
        _tc_y0_body,
        grid=(_NBLK,),
        in_specs=[
            pl.BlockSpec((8, H), lambda i: (jnp.minimum(i, _LAST), 0)),
            pl.BlockSpec((H, H), lambda i: (0, 0)),
            pl.BlockSpec((2, 8, 16), lambda i: (0, jnp.minimum(i, _LAST), 0)),
        ],
        out_specs=pl.BlockSpec((8, H), lambda i: (i, 0)),
        out_shape=jax.ShapeDtypeStruct((NPAD, H), jnp.float32),
    )(x, W0, degP)


def _ln_leaky(S_ref, y_ref, deg_ref, b_ref, g_ref, be_ref):
    dinv = _dinv_blk(deg_ref)
    S = S_ref[0] + S_ref[1]
    conv = dinv * (S + y_ref[...]) + b_ref[...]
    mu = jnp.mean(conv, axis=-1, keepdims=True)
    var = jnp.mean((conv - mu) ** 2, axis=-1, keepdims=True)
    a = (conv - mu) * lax.rsqrt(var + 1e-5) * g_ref[...] + be_ref[...]
    return jnp.where(a > 0, a, 0.1 * a), dinv


def _tc_mid_body(S_ref, y_ref, deg_ref, b_ref, g_ref, be_ref, wn_ref, yo_ref):
    i = pl.program_id(0)
    h, dinv = _ln_leaky(S_ref, y_ref, deg_ref, b_ref, g_ref, be_ref)
    yn = jnp.dot(h, wn_ref[...], preferred_element_type=jnp.float32,
                 precision=lax.Precision.HIGHEST) * dinv
    yo_ref[...] = jnp.where(i <= _LAST, yn, 0.0)


def _tc_mid(SP, y, degP, b, g, be, Wn):
    return pl.pallas_call(
        _tc_mid_body,
        grid=(_NBLK,),
        in_specs=[
            pl.BlockSpec((2, 8, H), lambda i: (0, jnp.minimum(i, _LAST), 0)),
            pl.BlockSpec((8, H), lambda i: (i, 0)),
            pl.BlockSpec((2, 8, 16), lambda i: (0, jnp.minimum(i, _LAST), 0)),
            pl.BlockSpec((1, H), lambda i: (0, 0)),
            pl.BlockSpec((1, H), lambda i: (0, 0)),
            pl.BlockSpec((1, H), lambda i: (0, 0)),
            pl.BlockSpec((H, H), lambda i: (0, 0)),
        ],
        out_specs=pl.BlockSpec((8, H), lambda i: (i, 0)),
        out_shape=jax.ShapeDtypeStruct((NPAD, H), jnp.float32),
    )(SP, y, degP, b, g, be, Wn)


def _tc_last_body(S_ref, y_ref, deg_ref, b_ref, g_ref, be_ref, h_ref):
    i = pl.program_id(0)
    h, _ = _ln_leaky(S_ref, y_ref, deg_ref, b_ref, g_ref, be_ref)
    h_ref[...] = jnp.where(i <= _LAST, h, 0.0)


def _tc_last(SP, y, degP, b, g, be):
    return pl.pallas_call(
        _tc_last_body,
        grid=(_NBLK,),
        in_specs=[
            pl.BlockSpec((2, 8, H), lambda i: (0, jnp.minimum(i, _LAST), 0)),
            pl.BlockSpec((8, H), lambda i: (i, 0)),
            pl.BlockSpec((2, 8, 16), lambda i: (0, jnp.minimum(i, _LAST), 0)),
            pl.BlockSpec((1, H), lambda i: (0, 0)),
            pl.BlockSpec((1, H), lambda i: (0, 0)),
            pl.BlockSpec((1, H), lambda i: (0, 0)),
        ],
        out_specs=pl.BlockSpec((8, H), lambda i: (i, 0)),
        out_shape=jax.ShapeDtypeStruct((NPAD, H), jnp.float32),
    )(SP, y, degP, b, g, be)


def _tc_final_body(mx_ref, sm_ref, cn_ref, wl1_ref, wl2_ref, bl_ref, o_ref):
    mx = jnp.max(mx_ref[...], axis=0)
    sm = jnp.sum(sm_ref[...], axis=0)
    cnt = jnp.sum(cn_ref[...], axis=0)
    mean = sm / jnp.maximum(cnt[:, 0:1], 1.0)
    o_ref[...] = (jnp.dot(mx, wl1_ref[...], preferred_element_type=jnp.float32,
                          precision=lax.Precision.HIGHEST)
                  + jnp.dot(mean, wl2_ref[...],
                            preferred_element_type=jnp.float32,
                            precision=lax.Precision.HIGHEST)
                  + bl_ref[...])


def _tc_final(mxP, smP, cntP, Wl1, Wl2, bl):
    return pl.pallas_call(
        _tc_final_body,
        grid=(1,),
        in_specs=[
            pl.BlockSpec((NW, G, H), lambda i: (0, 0, 0)),
            pl.BlockSpec((NW, G, H), lambda i: (0, 0, 0)),
            pl.BlockSpec((NW, G, 16), lambda i: (0, 0, 0)),
            pl.BlockSpec((H, H), lambda i: (0, 0)),
            pl.BlockSpec((H, H), lambda i: (0, 0)),
            pl.BlockSpec((1, H), lambda i: (0, 0)),
        ],
        out_specs=pl.BlockSpec((G, H), lambda i: (0, 0)),
        out_shape=jax.ShapeDtypeStruct((G, H), jnp.float32),
    )(mxP, smP, cntP, Wl1, Wl2, bl)


def kernel(x, edge_index, edge_attr, batch, W0, b0, W1, b1, W2, b2,
           g0, be0, g1, be1, g2, be2, Wl, bl):
    src = edge_index[0].astype(jnp.int32)
    dst = edge_index[1].astype(jnp.int32)
    pad = N + (jnp.arange(EPAD - E, dtype=jnp.int32) % 64)
    srcP = jnp.concatenate([src, pad]).reshape(NW, NCHUNK, CH)
    dstP = jnp.concatenate([dst, pad]).reshape(NW, NCHUNK, CH)
    batchP = jnp.concatenate(
        [batch.astype(jnp.int32), jnp.full((NPAD - N,), G, jnp.int32)])

    degP = _sc_deg(dstP)
    y = _tc_y0(x, W0, degP)
    S = _sc_msg(y, srcP, dstP)
    y = _tc_mid(S, y, degP, b0.reshape(1, H), g0.reshape(1, H),
                be0.reshape(1, H), W1)
    S = _sc_msg(y, srcP, dstP)
    y = _tc_mid(S, y, degP, b1.reshape(1, H), g1.reshape(1, H),
                be1.reshape(1, H), W2)
    S = _sc_msg(y, srcP, dstP)
    h3 = _tc_last(S, y, degP, b2.reshape(1, H), g2.reshape(1, H),
                  be2.reshape(1, H))
    mxP, smP, cntP = _sc_pool(h3.reshape(-1), batchP)
    out = _tc_final(mxP.reshape(NW, G, H), smP.reshape(NW, G, H),
                    cntP.reshape(NW, G, 16), Wl[:H], Wl[H:],
                    bl.reshape(1, H))
    return out


# speedup vs baseline: 21.9889x; 6.0195x over previous
import functools

import jax
import jax.numpy as jnp
from jax import lax
from jax.experimental import pallas as pl
from jax.experimental.pallas import tpu as pltpu
from jax.experimental.pallas import tpu_sc as plsc

N = 10000
E = 320000
H = 128
G = 64
NPAD = 10240
NW = 32
CH = 128
NCHUNK = 80
EPAD = NW * CH * NCHUNK
NACC = NPAD
ZPS = NACC // 16
RPT = NPAD // NW

_MESH = plsc.VectorSubcoreMesh(core_axis_name="c", subcore_axis_name="s")


def _sc_deg_body(dstP, degP, dst_v, ones_v, zbuf, acc, semd):
    c = lax.axis_index("c")
    s = lax.axis_index("s")
    w = c * 16 + s
    zeros16 = jnp.zeros((16,), jnp.float32)
    ones16 = jnp.ones((16,), jnp.float32)

    def init_row(i, _):
        zbuf[i, pl.ds(0, 16)] = zeros16
        ones_v[i, pl.ds(0, 16)] = ones16
        return 0

    lax.fori_loop(0, CH, init_row, 0)
    for t in range(ZPS // CH):
        pltpu.sync_copy(zbuf, acc.at[pl.ds(s * ZPS + t * CH, CH)])
    plsc.subcore_barrier()
    pltpu.sync_copy(dstP.at[w], dst_v)

    def sb(b, _):
        def fire(j, _):
            pltpu.async_copy(ones_v, acc.at[dst_v.at[b * 16 + j]], semd,
                             add=True)
            return 0

        lax.fori_loop(0, 16, fire, 0)

        def drain(j, _):
            pltpu.make_async_copy(ones_v, acc.at[dst_v.at[0]], semd).wait()
            return 0

        lax.fori_loop(0, 16, drain, 0)
        return 0

    lax.fori_loop(0, NCHUNK // 16, sb, 0)
    plsc.subcore_barrier()
    for t in range(ZPS // CH):
        pltpu.sync_copy(acc.at[pl.ds(s * ZPS + t * CH, CH)],
                        degP.at[c].at[pl.ds(s * ZPS + t * CH, CH)])


def _sc_deg(dstP):
    return pl.kernel(
        _sc_deg_body,
        out_type=jax.ShapeDtypeStruct((2, NPAD, 16), jnp.float32),
        mesh=_MESH,
        scratch_types=[
            pltpu.VMEM((NCHUNK, CH), jnp.int32),
            pltpu.VMEM((CH, 16), jnp.float32),
            pltpu.VMEM((CH, 16), jnp.float32),
            pltpu.VMEM_SHARED((NACC, 16), jnp.float32),
            pltpu.SemaphoreType.DMA,
        ],
    )(dstP)


def _sc_msg_body(yP, srcP, dstP, SP, dst_v, sidx0, sidx1, gbuf0, gbuf1,
                 acc, semg0, semg1, semi0, semi1):
    c = lax.axis_index("c")
    s = lax.axis_index("s")
    w = c * 16 + s
    zeros16 = jnp.zeros((16,), jnp.float32)

    def zrow(i, _):
        gbuf0[i // 8, pl.ds((i % 8) * 16, 16)] = zeros16
        return 0

    lax.fori_loop(0, CH * H // 16, zrow, 0)
    for t in range(ZPS // CH):
        pltpu.sync_copy(gbuf0, acc.at[pl.ds(s * ZPS + t * CH, CH)])
    plsc.subcore_barrier()
    pltpu.sync_copy(dstP.at[w], dst_v)
    pltpu.sync_copy(srcP.at[w].at[0], sidx0)
    pltpu.async_copy(yP.at[sidx0], gbuf0, semg0)
    pltpu.async_copy(srcP.at[w].at[1], sidx1, semi1)

    def body(i, _):
        j0 = 2 * i
        j1 = j0 + 1
        more = i < NCHUNK // 2 - 1
        pltpu.make_async_copy(yP.at[sidx0], gbuf0, semg0).wait()
        pltpu.make_async_copy(srcP.at[w].at[0], sidx1, semi1).wait()
        pltpu.async_copy(yP.at[sidx1], gbuf1, semg1)

        @pl.when(more)
        def _():
            pltpu.async_copy(srcP.at[w].at[j0 + 2], sidx0, semi0)

        pltpu.sync_copy(gbuf0, acc.at[dst_v.at[j0]], add=True)
        pltpu.make_async_copy(yP.at[sidx1], gbuf1, semg1).wait()

        @pl.when(more)
        def _():
            pltpu.make_async_copy(srcP.at[w].at[0], sidx0, semi0).wait()
            pltpu.async_copy(yP.at[sidx0], gbuf0, semg0)
            pltpu.async_copy(srcP.at[w].at[j1 + 2], sidx1, semi1)

        pltpu.sync_copy(gbuf1, acc.at[dst_v.at[j1]], add=True)
        return 0

    lax.fori_loop(0, NCHUNK // 2, body, 0)
    plsc.subcore_barrier()
    for t in range(ZPS // CH):
        pltpu.sync_copy(acc.at[pl.ds(s * ZPS + t * CH, CH)],
                        SP.at[c].at[pl.ds(s * ZPS + t * CH, CH)])


def _sc_msg(yP, srcP, dstP):
    return pl.kernel(
        _sc_msg_body,
        out_type=jax.ShapeDtypeStruct((2, NPAD, H), jnp.float32),
        mesh=_MESH,
        scratch_types=[
            pltpu.VMEM((NCHUNK, CH), jnp.int32),
            pltpu.VMEM((CH,), jnp.int32),
            pltpu.VMEM((CH,), jnp.int32),
            pltpu.VMEM((CH, H), jnp.float32),
            pltpu.VMEM((CH, H), jnp.float32),
            pltpu.VMEM_SHARED((NACC, H), jnp.float32),
            pltpu.SemaphoreType.DMA,
            pltpu.SemaphoreType.DMA,
            pltpu.SemaphoreType.DMA,
            pltpu.SemaphoreType.DMA,
        ],
    )(yP, srcP, dstP)


def _sc_pool_body(hflat, batchP, mxP, smP, cntP, hbuf, bbuf, mxl, sml, cnl):
    c = lax.axis_index("c")
    s = lax.axis_index("s")
    w = c * 16 + s
    zeros16 = jnp.zeros((16,), jnp.float32)
    ninf16 = jnp.full((16,), -jnp.inf, jnp.float32)
    ones16 = jnp.ones((16,), jnp.float32)

    def init_mx(i, _):
        mxl[pl.ds(i * 16, 16)] = ninf16
        sml[pl.ds(i * 16, 16)] = zeros16
        return 0

    lax.fori_loop(0, (G + 1) * H // 16, init_mx, 0)

    def init_cn(i, _):
        cnl[pl.ds(i * 16, 16)] = zeros16
        return 0

    lax.fori_loop(0, G + 1, init_cn, 0)

    pltpu.sync_copy(hflat.at[pl.ds(w * RPT * H, RPT * H)], hbuf)
    pltpu.sync_copy(batchP.at[pl.ds(w * RPT, RPT)], bbuf.at[pl.ds(0, RPT)])

    def row(r, _):
        b = bbuf[pl.ds(r, 16)][0]
        hb = r * H
        mb = b * H
        for k in range(H // 16):
            hv = hbuf[pl.ds(hb + k * 16, 16)]
            mv = mxl[pl.ds(mb + k * 16, 16)]
            mxl[pl.ds(mb + k * 16, 16)] = jnp.maximum(mv, hv)
            sv = sml[pl.ds(mb + k * 16, 16)]
            sml[pl.ds(mb + k * 16, 16)] = sv + hv
        cb = b * 16
        cnl[pl.ds(cb, 16)] = cnl[pl.ds(cb, 16)] + ones16
        return 0

    lax.fori_loop(0, RPT, row, 0)
    pltpu.sync_copy(mxl.at[pl.ds(0, G * H)], mxP.at[w])
    pltpu.sync_copy(sml.at[pl.ds(0, G * H)], smP.at[w])
    pltpu.sync_copy(cnl.at[pl.ds(0, G * 16)], cntP.at[w])


def _sc_pool(hflat, batchP):
    return pl.kernel(
        _sc_pool_body,
        out_type=(
            jax.ShapeDtypeStruct((NW, G * H), jnp.float32),
            jax.ShapeDtypeStruct((NW, G * H), jnp.float32),
            jax.ShapeDtypeStruct((NW, G * 16), jnp.float32),
        ),
        mesh=_MESH,
        scratch_types=[
            pltpu.VMEM((RPT * H,), jnp.float32),
            pltpu.VMEM((RPT + 16,), jnp.int32),
            pltpu.VMEM(((G + 1) * H,), jnp.float32),
            pltpu.VMEM(((G + 1) * H,), jnp.float32),
            pltpu.VMEM(((G + 1) * 16,), jnp.float32),
        ],
    )(hflat, batchP)


BR = 512
_NBLK = NPAD // BR


def _row_mask(i):
    ridx = i * BR + lax.broadcasted_iota(jnp.int32, (BR, 1), 0)
    return ridx < N


def _dinv_blk(deg_ref):
    deg = deg_ref[0] + deg_ref[1]
    return lax.rsqrt(deg[:, 0:1] + 1.0)


def _tc_y0_body(x_ref, w_ref, deg_ref, y_ref):
    i = pl.program_id(0)
    dinv = _dinv_blk(deg_ref)
    xw = jnp.dot(x_ref[...], w_ref[...], preferred_element_type=jnp.float32,
                 precision=lax.Precision.HIGHEST)
    y_ref[...] = jnp.where(_row_mask(i), xw * dinv, 0.0)


def _tc_y0(xP, W0, degP):
    return pl.pallas_call(
        _tc_y0_body,
        grid=(_NBLK,),
        in_specs=[
            pl.BlockSpec((BR, H), lambda i: (i, 0)),
            pl.BlockSpec((H, H), lambda i: (0, 0)),
            pl.BlockSpec((2, BR, 16), lambda i: (0, i, 0)),
        ],
        out_specs=pl.BlockSpec((BR, H), lambda i: (i, 0)),
        out_shape=jax.ShapeDtypeStruct((NPAD, H), jnp.float32),
    )(xP, W0, degP)


def _ln_leaky(S_ref, y_ref, deg_ref, b_ref, g_ref, be_ref):
    dinv = _dinv_blk(deg_ref)
    S = S_ref[0] + S_ref[1]
    conv = dinv * (S + y_ref[...]) + b_ref[...]
    mu = jnp.mean(conv, axis=-1, keepdims=True)
    var = jnp.mean((conv - mu) ** 2, axis=-1, keepdims=True)
    a = (conv - mu) * lax.rsqrt(var + 1e-5) * g_ref[...] + be_ref[...]
    return jnp.where(a > 0, a, 0.1 * a), dinv


def _tc_mid_body(S_ref, y_ref, deg_ref, b_ref, g_ref, be_ref, wn_ref, yo_ref):
    i = pl.program_id(0)
    h, dinv = _ln_leaky(S_ref, y_ref, deg_ref, b_ref, g_ref, be_ref)
    yn = jnp.dot(h, wn_ref[...], preferred_element_type=jnp.float32,
                 precision=lax.Precision.HIGHEST) * dinv
    yo_ref[...] = jnp.where(_row_mask(i), yn, 0.0)


def _tc_mid(SP, y, degP, b, g, be, Wn):
    return pl.pallas_call(
        _tc_mid_body,
        grid=(_NBLK,),
        in_specs=[
            pl.BlockSpec((2, BR, H), lambda i: (0, i, 0)),
            pl.BlockSpec((BR, H), lambda i: (i, 0)),
            pl.BlockSpec((2, BR, 16), lambda i: (0, i, 0)),
            pl.BlockSpec((1, H), lambda i: (0, 0)),
            pl.BlockSpec((1, H), lambda i: (0, 0)),
            pl.BlockSpec((1, H), lambda i: (0, 0)),
            pl.BlockSpec((H, H), lambda i: (0, 0)),
        ],
        out_specs=pl.BlockSpec((BR, H), lambda i: (i, 0)),
        out_shape=jax.ShapeDtypeStruct((NPAD, H), jnp.float32),
    )(SP, y, degP, b, g, be, Wn)


def _tc_last_body(S_ref, y_ref, deg_ref, b_ref, g_ref, be_ref, h_ref):
    i = pl.program_id(0)
    h, _ = _ln_leaky(S_ref, y_ref, deg_ref, b_ref, g_ref, be_ref)
    h_ref[...] = jnp.where(_row_mask(i), h, 0.0)


def _tc_last(SP, y, degP, b, g, be):
    return pl.pallas_call(
        _tc_last_body,
        grid=(_NBLK,),
        in_specs=[
            pl.BlockSpec((2, BR, H), lambda i: (0, i, 0)),
            pl.BlockSpec((BR, H), lambda i: (i, 0)),
            pl.BlockSpec((2, BR, 16), lambda i: (0, i, 0)),
            pl.BlockSpec((1, H), lambda i: (0, 0)),
            pl.BlockSpec((1, H), lambda i: (0, 0)),
            pl.BlockSpec((1, H), lambda i: (0, 0)),
        ],
        out_specs=pl.BlockSpec((BR, H), lambda i: (i, 0)),
        out_shape=jax.ShapeDtypeStruct((NPAD, H), jnp.float32),
    )(SP, y, degP, b, g, be)


def _tc_final_body(mx_ref, sm_ref, cn_ref, wl1_ref, wl2_ref, bl_ref, o_ref):
    mx = jnp.max(mx_ref[...], axis=0)
    sm = jnp.sum(sm_ref[...], axis=0)
    cnt = jnp.sum(cn_ref[...], axis=0)
    mean = sm / jnp.maximum(cnt[:, 0:1], 1.0)
    o_ref[...] = (jnp.dot(mx, wl1_ref[...], preferred_element_type=jnp.float32,
                          precision=lax.Precision.HIGHEST)
                  + jnp.dot(mean, wl2_ref[...],
                            preferred_element_type=jnp.float32,
                            precision=lax.Precision.HIGHEST)
                  + bl_ref[...])


def _tc_final(mxP, smP, cntP, Wl1, Wl2, bl):
    return pl.pallas_call(
        _tc_final_body,
        grid=(1,),
        in_specs=[
            pl.BlockSpec((NW, G, H), lambda i: (0, 0, 0)),
            pl.BlockSpec((NW, G, H), lambda i: (0, 0, 0)),
            pl.BlockSpec((NW, G, 16), lambda i: (0, 0, 0)),
            pl.BlockSpec((H, H), lambda i: (0, 0)),
            pl.BlockSpec((H, H), lambda i: (0, 0)),
            pl.BlockSpec((1, H), lambda i: (0, 0)),
        ],
        out_specs=pl.BlockSpec((G, H), lambda i: (0, 0)),
        out_shape=jax.ShapeDtypeStruct((G, H), jnp.float32),
    )(mxP, smP, cntP, Wl1, Wl2, bl)


def kernel(x, edge_index, edge_attr, batch, W0, b0, W1, b1, W2, b2,
           g0, be0, g1, be1, g2, be2, Wl, bl):
    src = edge_index[0].astype(jnp.int32)
    dst = edge_index[1].astype(jnp.int32)
    pad = N + (jnp.arange(EPAD - E, dtype=jnp.int32) % 64)
    srcP = jnp.concatenate([src, pad]).reshape(NW, NCHUNK, CH)
    dstP = jnp.concatenate([dst, pad]).reshape(NW, NCHUNK, CH)
    batchP = jnp.concatenate(
        [batch.astype(jnp.int32), jnp.full((NPAD - N,), G, jnp.int32)])

    xP = jnp.concatenate([x, jnp.zeros((NPAD - N, H), jnp.float32)])
    degP = _sc_deg(dstP)
    y = _tc_y0(xP, W0, degP)
    S = _sc_msg(y, srcP, dstP)
    y = _tc_mid(S, y, degP, b0.reshape(1, H), g0.reshape(1, H),
                be0.reshape(1, H), W1)
    S = _sc_msg(y, srcP, dstP)
    y = _tc_mid(S, y, degP, b1.reshape(1, H), g1.reshape(1, H),
                be1.reshape(1, H), W2)
    S = _sc_msg(y, srcP, dstP)
    h3 = _tc_last(S, y, degP, b2.reshape(1, H), g2.reshape(1, H),
                  be2.reshape(1, H))
    mxP, smP, cntP = _sc_pool(h3.reshape(-1), batchP)
    out = _tc_final(mxP.reshape(NW, G, H), smP.reshape(NW, G, H),
                    cntP.reshape(NW, G, 16), Wl[:H], Wl[H:],
                    bl.reshape(1, H))
    return out


# speedup vs baseline: 22.3089x; 1.0146x over previous
import functools

import jax
import jax.numpy as jnp
from jax import lax
from jax.experimental import pallas as pl
from jax.experimental.pallas import tpu as pltpu
from jax.experimental.pallas import tpu_sc as plsc

N = 10000
E = 320000
H = 128
G = 64
NPAD = 10240
NW = 32
CH = 128
NCHUNK = 80
EPAD = NW * CH * NCHUNK
NACC = NPAD
ZPS = NACC // 16
RPT = NPAD // NW

_MESH = plsc.VectorSubcoreMesh(core_axis_name="c", subcore_axis_name="s")


def _sc_deg_body(dstP, degP, dst_v, ones_v, zbuf, acc, semd):
    c = lax.axis_index("c")
    s = lax.axis_index("s")
    w = c * 16 + s
    zeros16 = jnp.zeros((16,), jnp.float32)
    ones16 = jnp.ones((16,), jnp.float32)

    def init_row(i, _):
        zbuf[i, pl.ds(0, 16)] = zeros16
        ones_v[i, pl.ds(0, 16)] = ones16
        return 0

    lax.fori_loop(0, CH, init_row, 0)
    for t in range(ZPS // CH):
        pltpu.sync_copy(zbuf, acc.at[pl.ds(s * ZPS + t * CH, CH)])
    plsc.subcore_barrier()
    pltpu.sync_copy(dstP.at[w], dst_v)

    def sb(b, _):
        def fire(j, _):
            pltpu.async_copy(ones_v, acc.at[dst_v.at[b * 16 + j]], semd,
                             add=True)
            return 0

        lax.fori_loop(0, 16, fire, 0)

        def drain(j, _):
            pltpu.make_async_copy(ones_v, acc.at[dst_v.at[0]], semd).wait()
            return 0

        lax.fori_loop(0, 16, drain, 0)
        return 0

    lax.fori_loop(0, NCHUNK // 16, sb, 0)
    plsc.subcore_barrier()
    for t in range(ZPS // CH):
        pltpu.sync_copy(acc.at[pl.ds(s * ZPS + t * CH, CH)],
                        degP.at[c].at[pl.ds(s * ZPS + t * CH, CH)])


def _sc_deg(dstP):
    return pl.kernel(
        _sc_deg_body,
        out_type=jax.ShapeDtypeStruct((2, NPAD, 16), jnp.float32),
        mesh=_MESH,
        scratch_types=[
            pltpu.VMEM((NCHUNK, CH), jnp.int32),
            pltpu.VMEM((CH, 16), jnp.float32),
            pltpu.VMEM((CH, 16), jnp.float32),
            pltpu.VMEM_SHARED((NACC, 16), jnp.float32),
            pltpu.SemaphoreType.DMA,
        ],
    )(dstP)


def _sc_msg_body(yP, srcP, dstP, SP, dst_v, sidx0, sidx1, gbuf0, gbuf1,
                 acc, semg0, semg1, semi0, semi1):
    c = lax.axis_index("c")
    s = lax.axis_index("s")
    w = c * 16 + s
    zeros16 = jnp.zeros((16,), jnp.float32)

    def zrow(i, _):
        gbuf0[i // 8, pl.ds((i % 8) * 16, 16)] = zeros16
        return 0

    lax.fori_loop(0, CH * H // 16, zrow, 0)
    for t in range(ZPS // CH):
        pltpu.sync_copy(gbuf0, acc.at[pl.ds(s * ZPS + t * CH, CH)])
    plsc.subcore_barrier()
    pltpu.sync_copy(dstP.at[w], dst_v)
    pltpu.sync_copy(srcP.at[w].at[0], sidx0)
    pltpu.async_copy(yP.at[sidx0], gbuf0, semg0)
    pltpu.async_copy(srcP.at[w].at[1], sidx1, semi1)

    def body(i, _):
        j0 = 2 * i
        j1 = j0 + 1
        more = i < NCHUNK // 2 - 1
        pltpu.make_async_copy(yP.at[sidx0], gbuf0, semg0).wait()
        pltpu.make_async_copy(srcP.at[w].at[0], sidx1, semi1).wait()
        pltpu.async_copy(yP.at[sidx1], gbuf1, semg1)

        @pl.when(more)
        def _():
            pltpu.async_copy(srcP.at[w].at[j0 + 2], sidx0, semi0)

        pltpu.sync_copy(gbuf0, acc.at[dst_v.at[j0]], add=True)
        pltpu.make_async_copy(yP.at[sidx1], gbuf1, semg1).wait()

        @pl.when(more)
        def _():
            pltpu.make_async_copy(srcP.at[w].at[0], sidx0, semi0).wait()
            pltpu.async_copy(yP.at[sidx0], gbuf0, semg0)
            pltpu.async_copy(srcP.at[w].at[j1 + 2], sidx1, semi1)

        pltpu.sync_copy(gbuf1, acc.at[dst_v.at[j1]], add=True)
        return 0

    lax.fori_loop(0, NCHUNK // 2, body, 0)
    plsc.subcore_barrier()
    for t in range(ZPS // CH):
        pltpu.sync_copy(acc.at[pl.ds(s * ZPS + t * CH, CH)],
                        SP.at[c].at[pl.ds(s * ZPS + t * CH, CH)])


def _sc_msg(yP, srcP, dstP):
    return pl.kernel(
        _sc_msg_body,
        out_type=jax.ShapeDtypeStruct((2, NPAD, H), jnp.float32),
        mesh=_MESH,
        scratch_types=[
            pltpu.VMEM((NCHUNK, CH), jnp.int32),
            pltpu.VMEM((CH,), jnp.int32),
            pltpu.VMEM((CH,), jnp.int32),
            pltpu.VMEM((CH, H), jnp.float32),
            pltpu.VMEM((CH, H), jnp.float32),
            pltpu.VMEM_SHARED((NACC, H), jnp.float32),
            pltpu.SemaphoreType.DMA,
            pltpu.SemaphoreType.DMA,
            pltpu.SemaphoreType.DMA,
            pltpu.SemaphoreType.DMA,
        ],
    )(yP, srcP, dstP)


def _sc_pool_body(hflat, batchP, mxP, smP, cntP, hbuf, bbuf, mxl, sml, cnl):
    c = lax.axis_index("c")
    s = lax.axis_index("s")
    w = c * 16 + s
    zeros16 = jnp.zeros((16,), jnp.float32)
    ninf16 = jnp.full((16,), -jnp.inf, jnp.float32)
    ones16 = jnp.ones((16,), jnp.float32)

    def init_mx(i, _):
        mxl[pl.ds(i * 16, 16)] = ninf16
        sml[pl.ds(i * 16, 16)] = zeros16
        return 0

    lax.fori_loop(0, (G + 1) * H // 16, init_mx, 0)

    def init_cn(i, _):
        cnl[pl.ds(i * 16, 16)] = zeros16
        return 0

    lax.fori_loop(0, G + 1, init_cn, 0)

    pltpu.sync_copy(hflat.at[pl.ds(w * RPT * H, RPT * H)], hbuf)
    pltpu.sync_copy(batchP.at[pl.ds(w * RPT, RPT)], bbuf.at[pl.ds(0, RPT)])

    def row(r, _):
        b = bbuf[pl.ds(r, 16)][0]
        hb = r * H
        mb = b * H
        for k in range(H // 16):
            hv = hbuf[pl.ds(hb + k * 16, 16)]
            mv = mxl[pl.ds(mb + k * 16, 16)]
            mxl[pl.ds(mb + k * 16, 16)] = jnp.maximum(mv, hv)
            sv = sml[pl.ds(mb + k * 16, 16)]
            sml[pl.ds(mb + k * 16, 16)] = sv + hv
        cb = b * 16
        cnl[pl.ds(cb, 16)] = cnl[pl.ds(cb, 16)] + ones16
        return 0

    lax.fori_loop(0, RPT, row, 0)
    pltpu.sync_copy(mxl.at[pl.ds(0, G * H)], mxP.at[w])
    pltpu.sync_copy(sml.at[pl.ds(0, G * H)], smP.at[w])
    pltpu.sync_copy(cnl.at[pl.ds(0, G * 16)], cntP.at[w])


def _sc_pool(hflat, batchP):
    return pl.kernel(
        _sc_pool_body,
        out_type=(
            jax.ShapeDtypeStruct((NW, G * H), jnp.float32),
            jax.ShapeDtypeStruct((NW, G * H), jnp.float32),
            jax.ShapeDtypeStruct((NW, G * 16), jnp.float32),
        ),
        mesh=_MESH,
        scratch_types=[
            pltpu.VMEM((RPT * H,), jnp.float32),
            pltpu.VMEM((RPT + 16,), jnp.int32),
            pltpu.VMEM(((G + 1) * H,), jnp.float32),
            pltpu.VMEM(((G + 1) * H,), jnp.float32),
            pltpu.VMEM(((G + 1) * 16,), jnp.float32),
        ],
    )(hflat, batchP)


BR = 512
_NBLK = NPAD // BR


def _row_mask(i):
    ridx = i * BR + lax.broadcasted_iota(jnp.int32, (BR, 1), 0)
    return ridx < N


def _dinv_blk(deg_ref):
    deg = deg_ref[0] + deg_ref[1]
    return lax.rsqrt(deg[:, 0:1] + 1.0)


def _tc_y0_body(x_ref, w_ref, deg_ref, y_ref):
    i = pl.program_id(0)
    dinv = _dinv_blk(deg_ref)
    xw = jnp.dot(x_ref[...], w_ref[...], preferred_element_type=jnp.float32,
                 precision=lax.Precision.DEFAULT)
    y_ref[...] = jnp.where(_row_mask(i), xw * dinv, 0.0)


def _tc_y0(xP, W0, degP):
    return pl.pallas_call(
        _tc_y0_body,
        grid=(_NBLK,),
        in_specs=[
            pl.BlockSpec((BR, H), lambda i: (i, 0)),
            pl.BlockSpec((H, H), lambda i: (0, 0)),
            pl.BlockSpec((2, BR, 16), lambda i: (0, i, 0)),
        ],
        out_specs=pl.BlockSpec((BR, H), lambda i: (i, 0)),
        out_shape=jax.ShapeDtypeStruct((NPAD, H), jnp.float32),
    )(xP, W0, degP)


def _ln_leaky(S_ref, y_ref, deg_ref, b_ref, g_ref, be_ref):
    dinv = _dinv_blk(deg_ref)
    S = S_ref[0] + S_ref[1]
    conv = dinv * (S + y_ref[...]) + b_ref[...]
    mu = jnp.mean(conv, axis=-1, keepdims=True)
    var = jnp.mean((conv - mu) ** 2, axis=-1, keepdims=True)
    a = (conv - mu) * lax.rsqrt(var + 1e-5) * g_ref[...] + be_ref[...]
    return jnp.where(a > 0, a, 0.1 * a), dinv


def _tc_mid_body(S_ref, y_ref, deg_ref, b_ref, g_ref, be_ref, wn_ref, yo_ref):
    i = pl.program_id(0)
    h, dinv = _ln_leaky(S_ref, y_ref, deg_ref, b_ref, g_ref, be_ref)
    yn = jnp.dot(h, wn_ref[...], preferred_element_type=jnp.float32,
                 precision=lax.Precision.DEFAULT) * dinv
    yo_ref[...] = jnp.where(_row_mask(i), yn, 0.0)


def _tc_mid(SP, y, degP, b, g, be, Wn):
    return pl.pallas_call(
        _tc_mid_body,
        grid=(_NBLK,),
        in_specs=[
            pl.BlockSpec((2, BR, H), lambda i: (0, i, 0)),
            pl.BlockSpec((BR, H), lambda i: (i, 0)),
            pl.BlockSpec((2, BR, 16), lambda i: (0, i, 0)),
            pl.BlockSpec((1, H), lambda i: (0, 0)),
            pl.BlockSpec((1, H), lambda i: (0, 0)),
            pl.BlockSpec((1, H), lambda i: (0, 0)),
            pl.BlockSpec((H, H), lambda i: (0, 0)),
        ],
        out_specs=pl.BlockSpec((BR, H), lambda i: (i, 0)),
        out_shape=jax.ShapeDtypeStruct((NPAD, H), jnp.float32),
    )(SP, y, degP, b, g, be, Wn)


def _tc_last_body(S_ref, y_ref, deg_ref, b_ref, g_ref, be_ref, h_ref):
    i = pl.program_id(0)
    h, _ = _ln_leaky(S_ref, y_ref, deg_ref, b_ref, g_ref, be_ref)
    h_ref[...] = jnp.where(_row_mask(i), h, 0.0)


def _tc_last(SP, y, degP, b, g, be):
    return pl.pallas_call(
        _tc_last_body,
        grid=(_NBLK,),
        in_specs=[
            pl.BlockSpec((2, BR, H), lambda i: (0, i, 0)),
            pl.BlockSpec((BR, H), lambda i: (i, 0)),
            pl.BlockSpec((2, BR, 16), lambda i: (0, i, 0)),
            pl.BlockSpec((1, H), lambda i: (0, 0)),
            pl.BlockSpec((1, H), lambda i: (0, 0)),
            pl.BlockSpec((1, H), lambda i: (0, 0)),
        ],
        out_specs=pl.BlockSpec((BR, H), lambda i: (i, 0)),
        out_shape=jax.ShapeDtypeStruct((NPAD, H), jnp.float32),
    )(SP, y, degP, b, g, be)


def _tc_final_body(mx_ref, sm_ref, cn_ref, wl1_ref, wl2_ref, bl_ref, o_ref):
    mx = jnp.max(mx_ref[...], axis=0)
    sm = jnp.sum(sm_ref[...], axis=0)
    cnt = jnp.sum(cn_ref[...], axis=0)
    mean = sm / jnp.maximum(cnt[:, 0:1], 1.0)
    o_ref[...] = (jnp.dot(mx, wl1_ref[...], preferred_element_type=jnp.float32,
                          precision=lax.Precision.DEFAULT)
                  + jnp.dot(mean, wl2_ref[...],
                            preferred_element_type=jnp.float32,
                            precision=lax.Precision.DEFAULT)
                  + bl_ref[...])


def _tc_final(mxP, smP, cntP, Wl1, Wl2, bl):
    return pl.pallas_call(
        _tc_final_body,
        grid=(1,),
        in_specs=[
            pl.BlockSpec((NW, G, H), lambda i: (0, 0, 0)),
            pl.BlockSpec((NW, G, H), lambda i: (0, 0, 0)),
            pl.BlockSpec((NW, G, 16), lambda i: (0, 0, 0)),
            pl.BlockSpec((H, H), lambda i: (0, 0)),
            pl.BlockSpec((H, H), lambda i: (0, 0)),
            pl.BlockSpec((1, H), lambda i: (0, 0)),
        ],
        out_specs=pl.BlockSpec((G, H), lambda i: (0, 0)),
        out_shape=jax.ShapeDtypeStruct((G, H), jnp.float32),
    )(mxP, smP, cntP, Wl1, Wl2, bl)


def kernel(x, edge_index, edge_attr, batch, W0, b0, W1, b1, W2, b2,
           g0, be0, g1, be1, g2, be2, Wl, bl):
    src = edge_index[0].astype(jnp.int32)
    dst = edge_index[1].astype(jnp.int32)
    pad = N + (jnp.arange(EPAD - E, dtype=jnp.int32) % 64)
    srcP = jnp.concatenate([src, pad]).reshape(NW, NCHUNK, CH)
    dstP = jnp.concatenate([dst, pad]).reshape(NW, NCHUNK, CH)
    batchP = jnp.concatenate(
        [batch.astype(jnp.int32), jnp.full((NPAD - N,), G, jnp.int32)])

    xP = jnp.concatenate([x, jnp.zeros((NPAD - N, H), jnp.float32)])
    degP = _sc_deg(dstP)
    y = _tc_y0(xP, W0, degP)
    S = _sc_msg(y, srcP, dstP)
    y = _tc_mid(S, y, degP, b0.reshape(1, H), g0.reshape(1, H),
                be0.reshape(1, H), W1)
    S = _sc_msg(y, srcP, dstP)
    y = _tc_mid(S, y, degP, b1.reshape(1, H), g1.reshape(1, H),
                be1.reshape(1, H), W2)
    S = _sc_msg(y, srcP, dstP)
    h3 = _tc_last(S, y, degP, b2.reshape(1, H), g2.reshape(1, H),
                  be2.reshape(1, H))
    mxP, smP, cntP = _sc_pool(h3.reshape(-1), batchP)
    out = _tc_final(mxP.reshape(NW, G, H), smP.reshape(NW, G, H),
                    cntP.reshape(NW, G, 16), Wl[:H], Wl[H:],
                    bl.reshape(1, H))
    return out
